# Initial kernel scaffold; baseline (speedup 1.0000x reference)
import jax
import jax.numpy as jnp
from jax.experimental import pallas as pl


def kernel(seq1, seq2, adj, sparse, msk, samp_bias1, samp_bias2, W_fc, gcn_bias, prelu_a, W_bil, b_bil):
    raise NotImplementedError("write your pallas kernel here")


# speedup vs baseline: 1.8639x; 1.8639x over previous
import jax
import jax.numpy as jnp
from jax.experimental import pallas as pl
from jax.experimental.pallas import tpu as pltpu


def _dgi_body(adj_ref, s1_ref, s2_ref, wfcT_ref, b2_ref, a_ref, mcol_ref,
              wbilT_ref, bb_ref, sb_ref, out_ref,
              fts, hbuf, rsum, cnt):
    r = pl.program_id(0)
    nr = pl.num_programs(0)
    n, two_h = fts.shape
    h_dim = two_h // 2
    br = n // nr

    @pl.when(r == 0)
    def _project():
        w = wfcT_ref[...].astype(jnp.bfloat16)
        f1 = jnp.dot(s1_ref[...].astype(jnp.bfloat16), w,
                     preferred_element_type=jnp.float32)
        f2 = jnp.dot(s2_ref[...].astype(jnp.bfloat16), w,
                     preferred_element_type=jnp.float32)
        fts[:, :h_dim] = f1.astype(jnp.bfloat16)
        fts[:, h_dim:] = f2.astype(jnp.bfloat16)

    hblk = jnp.dot(adj_ref[...].astype(jnp.bfloat16), fts[...],
                   preferred_element_type=jnp.float32)
    hblk = hblk + b2_ref[...]
    alpha = a_ref[0, 0]
    hblk = jnp.where(hblk >= 0, hblk, alpha * hblk)
    hbuf[pl.ds(r * br, br), :] = hblk.astype(jnp.bfloat16)

    mcol = mcol_ref[...]
    part = jax.lax.dot_general(
        mcol, hblk[:, :h_dim], (((0,), (0,)), ((), ())),
        preferred_element_type=jnp.float32)
    pcnt = jnp.sum(mcol)

    @pl.when(r == 0)
    def _init_rsum():
        rsum[...] = part
        cnt[0, 0] = pcnt

    @pl.when(r != 0)
    def _add_rsum():
        rsum[...] += part
        cnt[0, 0] += pcnt

    @pl.when(r == nr - 1)
    def _scores():
        c = jax.nn.sigmoid(rsum[...] / cnt[0, 0])
        w_row = jnp.dot(c, wbilT_ref[...],
                        preferred_element_type=jnp.float32)
        w16 = w_row.astype(jnp.bfloat16)
        z = jnp.zeros((1, h_dim), jnp.bfloat16)
        w2 = jnp.concatenate(
            [jnp.concatenate([w16, z], axis=1),
             jnp.concatenate([z, w16], axis=1)], axis=0)
        sc = jax.lax.dot_general(
            w2, hbuf[...], (((1,), (1,)), ((), ())),
            preferred_element_type=jnp.float32)
        out_ref[...] = sc + bb_ref[0, 0] + sb_ref[...]


def kernel(seq1, seq2, adj, sparse, msk, samp_bias1, samp_bias2,
           W_fc, gcn_bias, prelu_a, W_bil, b_bil):
    del sparse
    _, n, d = seq1.shape
    h_dim = W_fc.shape[0]
    br = 400
    nr = n // br

    s1 = seq1.reshape(n, d)
    s2 = seq2.reshape(n, d)
    adj2 = adj.reshape(n, n)
    wfcT = W_fc.T
    b2 = jnp.concatenate([gcn_bias, gcn_bias])[None, :]
    a11 = prelu_a.reshape(1, 1)
    mcol = msk.reshape(n, 1)
    wbilT = W_bil[0].T
    bb = b_bil.reshape(1, 1)
    sb = jnp.concatenate([samp_bias1, samp_bias2], axis=0)

    out = pl.pallas_call(
        _dgi_body,
        grid=(nr,),
        in_specs=[
            pl.BlockSpec((br, n), lambda r: (r, 0)),
            pl.BlockSpec((n, d), lambda r: (0, 0)),
            pl.BlockSpec((n, d), lambda r: (0, 0)),
            pl.BlockSpec((d, h_dim), lambda r: (0, 0)),
            pl.BlockSpec((1, 2 * h_dim), lambda r: (0, 0)),
            pl.BlockSpec((1, 1), lambda r: (0, 0)),
            pl.BlockSpec((br, 1), lambda r: (r, 0)),
            pl.BlockSpec((h_dim, h_dim), lambda r: (0, 0)),
            pl.BlockSpec((1, 1), lambda r: (0, 0)),
            pl.BlockSpec((2, n), lambda r: (0, 0)),
        ],
        out_specs=pl.BlockSpec((2, n), lambda r: (0, 0)),
        out_shape=jax.ShapeDtypeStruct((2, n), jnp.float32),
        scratch_shapes=[
            pltpu.VMEM((n, 2 * h_dim), jnp.bfloat16),
            pltpu.VMEM((n, 2 * h_dim), jnp.bfloat16),
            pltpu.VMEM((1, h_dim), jnp.float32),
            pltpu.SMEM((1, 1), jnp.float32),
        ],
        compiler_params=pltpu.CompilerParams(
            dimension_semantics=("arbitrary",),
        ),
    )(adj2, s1, s2, wfcT, b2, a11, mcol, wbilT, bb, sb)

    return out.reshape(1, 2 * n)
